# baseline (speedup 1.0000x reference)
import functools

import jax
import jax.numpy as jnp
from jax import lax
from jax.experimental import pallas as pl
from jax.experimental.pallas import tpu as pltpu
from jax.experimental.pallas import tpu_sc as plsc

_N = 100000
_S = 64
_C = 800
_K = _N // _C
_NW = 32
_T = (_K + _NW - 1) // _NW
_G = _C // 16


def _body(idx_hbm, out_hbm, idx_v, buf_v):
    c = lax.axis_index("c")
    s = lax.axis_index("s")
    w = s * 2 + c

    lanes = lax.iota(jnp.int32, 16)
    row_off = lanes * _S
    ones = jnp.full((16,), 1.0, jnp.float32)
    zeros = jnp.zeros((16,), jnp.float32)

    def zero_loop(k, carry):
        buf_v[pl.ds(k * 16, 16)] = zeros
        return carry

    lax.fori_loop(0, (_C * _S) // 16, zero_loop, None)

    for t in range(_T):
        cid = w + _NW * t

        @pl.when(cid < _K)
        def _process():
            rowbase = cid * _C
            pltpu.sync_copy(idx_hbm.at[pl.ds(rowbase, _C)], idx_v)

            def fill(i, carry):
                ids = idx_v[pl.ds(i * 16, 16)]
                pos = i * (16 * _S) + row_off + ids
                plsc.store_scatter(buf_v, [pos], ones)
                return carry

            lax.fori_loop(0, _G, fill, None)

            pltpu.sync_copy(buf_v, out_hbm.at[pl.ds(rowbase * _S, _C * _S)])

            def unfill(i, carry):
                ids = idx_v[pl.ds(i * 16, 16)]
                pos = i * (16 * _S) + row_off + ids
                plsc.store_scatter(buf_v, [pos], zeros)
                return carry

            lax.fori_loop(0, _G, unfill, None)


@jax.jit
def _onehot_sc(species_index):
    mesh = plsc.VectorSubcoreMesh(core_axis_name="c", subcore_axis_name="s")
    f = pl.kernel(
        _body,
        out_type=jax.ShapeDtypeStruct((_N * _S,), jnp.float32),
        mesh=mesh,
        compiler_params=pltpu.CompilerParams(needs_layout_passes=False),
        scratch_types=[
            pltpu.VMEM((_C,), jnp.int32),
            pltpu.VMEM((_C * _S,), jnp.float32),
        ],
    )
    return f(species_index)


def kernel(species_index, pos):
    out = _onehot_sc(species_index)
    return out.reshape(species_index.shape[0], _S)


# speedup vs baseline: 1.1534x; 1.1534x over previous
import jax
import jax.numpy as jnp
from jax import lax
from jax.experimental import pallas as pl
from jax.experimental.pallas import tpu as pltpu
from jax.experimental.pallas import tpu_sc as plsc

_N = 100000
_S = 64
_C = 800
_K = _N // _C
_NW = 32
_T = (_K + _NW - 1) // _NW
_G = _C // 16


def _body(idx_hbm, out_hbm, idx_v, buf_v, isems, osems):
    c = lax.axis_index("c")
    s = lax.axis_index("s")
    w = s * 2 + c

    lanes = lax.iota(jnp.int32, 16)
    row_off = lanes * _S
    ones = jnp.full((16,), 1.0, jnp.float32)
    zeros = jnp.zeros((16,), jnp.float32)

    cids = [jnp.minimum(w + _NW * t, _K - 1) for t in range(_T)]

    idx_dmas = []
    for t in range(_T):
        dma = pltpu.make_async_copy(
            idx_hbm.at[pl.ds(cids[t] * _C, _C)], idx_v[t], isems[t]
        )
        dma.start()
        idx_dmas.append(dma)

    def _zero(b):
        def zloop(k, carry):
            base = k * 128
            for u in range(8):
                b[pl.ds(base + u * 16, 16)] = zeros
            return carry

        lax.fori_loop(0, (_C * _S) // 128, zloop, None)

    _zero(buf_v[0])
    _zero(buf_v[1])

    def _scatter(buf, idx, val):
        def loop(i, carry):
            ids = idx[pl.ds(i * 16, 16)]
            pos = i * (16 * _S) + row_off + ids
            plsc.store_scatter(buf, [pos], val)
            return carry

        lax.fori_loop(0, _G, loop, None)

    out_dmas = [None, None]
    for t in range(_T):
        b = t % 2
        if t >= 2:
            out_dmas[b].wait()
            _scatter(buf_v[b], idx_v[t - 2], zeros)
        idx_dmas[t].wait()
        _scatter(buf_v[b], idx_v[t], ones)
        out_dmas[b] = pltpu.make_async_copy(
            buf_v[b], out_hbm.at[pl.ds(cids[t] * _C * _S, _C * _S)], osems[b]
        )
        out_dmas[b].start()

    out_dmas[0].wait()
    out_dmas[1].wait()


@jax.jit
def _onehot_sc(species_index):
    mesh = plsc.VectorSubcoreMesh(core_axis_name="c", subcore_axis_name="s")
    f = pl.kernel(
        _body,
        out_type=jax.ShapeDtypeStruct((_N * _S,), jnp.float32),
        mesh=mesh,
        compiler_params=pltpu.CompilerParams(needs_layout_passes=False),
        scratch_types=[
            [pltpu.VMEM((_C,), jnp.int32) for _ in range(_T)],
            [pltpu.VMEM((_C * _S,), jnp.float32) for _ in range(2)],
            [pltpu.SemaphoreType.DMA for _ in range(_T)],
            [pltpu.SemaphoreType.DMA for _ in range(2)],
        ],
    )
    return f(species_index)


def kernel(species_index, pos):
    out = _onehot_sc(species_index)
    return out.reshape(species_index.shape[0], _S)


# speedup vs baseline: 1.4655x; 1.2706x over previous
import jax
import jax.numpy as jnp
from jax import lax
from jax.experimental import pallas as pl
from jax.experimental.pallas import tpu as pltpu
from jax.experimental.pallas import tpu_sc as plsc

_N = 100000
_S = 64
_C = 400
_K = _N // _C
_NW = 32
_T = (_K + _NW - 1) // _NW
_G = _C // 16


def _body(idx_hbm, out_hbm, idx_v, buf_v, isems, osems):
    c = lax.axis_index("c")
    s = lax.axis_index("s")
    w = s * 2 + c

    lanes = lax.iota(jnp.int32, 16)
    ones = jnp.full((16,), 1.0, jnp.float32)
    zeros = jnp.zeros((16,), jnp.float32)

    cids = [jnp.minimum(w + _NW * t, _K - 1) for t in range(_T)]

    idx_dmas = []
    for t in range(_T):
        dma = pltpu.make_async_copy(
            idx_hbm.at[pl.ds(cids[t] * _C, _C)], idx_v[t], isems[t]
        )
        dma.start()
        idx_dmas.append(dma)

    def _zero(b):
        def zloop(r, carry):
            rows = lanes * 0 + r
            for u in range(_S // 16):
                plsc.store_scatter(b, [rows, lanes + 16 * u], zeros)
            return carry

        lax.fori_loop(0, _C, zloop, None)

    _zero(buf_v[0])
    _zero(buf_v[1])

    def _scatter(buf, idx, val):
        def loop(i, carry):
            ids = idx[pl.ds(i * 16, 16)]
            plsc.store_scatter(buf, [i * 16 + lanes, ids], val)
            return carry

        lax.fori_loop(0, _G, loop, None)

    out_dmas = [None, None]
    for t in range(_T):
        b = t % 2
        if t >= 2:
            out_dmas[b].wait()
            _scatter(buf_v[b], idx_v[t - 2], zeros)
        idx_dmas[t].wait()
        _scatter(buf_v[b], idx_v[t], ones)
        out_dmas[b] = pltpu.make_async_copy(
            buf_v[b], out_hbm.at[pl.ds(cids[t] * _C, _C)], osems[b]
        )
        out_dmas[b].start()

    out_dmas[0].wait()
    out_dmas[1].wait()


@jax.jit
def _onehot_sc(species_index):
    mesh = plsc.VectorSubcoreMesh(core_axis_name="c", subcore_axis_name="s")
    f = pl.kernel(
        _body,
        out_type=jax.ShapeDtypeStruct((_N, _S), jnp.float32),
        mesh=mesh,
        compiler_params=pltpu.CompilerParams(needs_layout_passes=False),
        scratch_types=[
            [pltpu.VMEM((_C,), jnp.int32) for _ in range(_T)],
            [pltpu.VMEM((_C, _S), jnp.float32) for _ in range(2)],
            [pltpu.SemaphoreType.DMA for _ in range(_T)],
            [pltpu.SemaphoreType.DMA for _ in range(2)],
        ],
    )
    return f(species_index)


def kernel(species_index, pos):
    return _onehot_sc(species_index)


# speedup vs baseline: 1.4775x; 1.0082x over previous
import jax
import jax.numpy as jnp
from jax import lax
from jax.experimental import pallas as pl
from jax.experimental.pallas import tpu as pltpu
from jax.experimental.pallas import tpu_sc as plsc

_N = 100000
_S = 64
_C = 400
_K = _N // _C
_NW = 32
_T = (_K + _NW - 1) // _NW
_G = _C // 16


def _body(idx_hbm, out_hbm, idx_v, buf_v, isems, osems):
    c = lax.axis_index("c")
    s = lax.axis_index("s")
    w = s * 2 + c

    lanes = lax.iota(jnp.int32, 16)
    ones = jnp.full((16,), 1.0, jnp.float32)
    zeros = jnp.zeros((16,), jnp.float32)

    cids = [jnp.minimum(w + _NW * t, _K - 1) for t in range(_T)]

    idx_dmas = []
    for t in range(_T):
        dma = pltpu.make_async_copy(
            idx_hbm.at[pl.ds(cids[t] * _C, _C)], idx_v[t], isems[t]
        )
        dma.start()
        idx_dmas.append(dma)

    def _zero(b):
        def zloop(r, carry):
            rows = lanes * 0 + r
            for u in range(_S // 16):
                plsc.store_scatter(b, [rows, lanes + 16 * u], zeros)
            return carry

        lax.fori_loop(0, _C, zloop, None)

    _zero(buf_v[0])
    _zero(buf_v[1])

    def _scatter(buf, idx, val):
        def loop(i, carry):
            ids = idx[pl.ds(i * 16, 16)]
            plsc.store_scatter(buf, [i * 16 + lanes, ids], val)
            return carry

        lax.fori_loop(0, _G, loop, None)

    out_dmas = [None, None]
    for t in range(_T):
        b = t % 2
        if t >= 2:
            out_dmas[b].wait()
            _scatter(buf_v[b], idx_v[t - 2], zeros)
        idx_dmas[t].wait()
        _scatter(buf_v[b], idx_v[t], ones)
        out_dmas[b] = pltpu.make_async_copy(
            buf_v[b], out_hbm.at[pl.ds(cids[t] * _C, _C)], osems[b]
        )
        out_dmas[b].start()

    out_dmas[0].wait()
    out_dmas[1].wait()


@jax.jit
def _onehot_sc(species_index):
    mesh = plsc.VectorSubcoreMesh(core_axis_name="c", subcore_axis_name="s")
    f = pl.kernel(
        _body,
        out_type=jax.ShapeDtypeStruct((_N, _S), jnp.float32),
        mesh=mesh,
        compiler_params=pltpu.CompilerParams(
            needs_layout_passes=False, use_tc_tiling_on_sc=True
        ),
        scratch_types=[
            [pltpu.VMEM((_C,), jnp.int32) for _ in range(_T)],
            [pltpu.VMEM((_C, _S), jnp.float32) for _ in range(2)],
            [pltpu.SemaphoreType.DMA for _ in range(_T)],
            [pltpu.SemaphoreType.DMA for _ in range(2)],
        ],
    )
    return f(species_index)


def kernel(species_index, pos):
    return _onehot_sc(species_index)
